# baseline (speedup 1.0000x reference)
import functools

import jax
import jax.numpy as jnp
from jax import lax
from jax.experimental import pallas as pl
from jax.experimental.pallas import tpu as pltpu
from jax.experimental.pallas import tpu_sc as plsc

_ROWS = 4 * 4096
_D = 1024
_K = _ROWS // 2
_BLK = 1024
_WAYS = 8

_NW = 32
_RPW = 64
_RSC = _NW * _RPW
_RTC = _ROWS - _RSC
_NBLK = _RTC // _BLK
_CH = 16
_L = 16


def _tc_loss_body(pred_ref, target_ref, loss_ref):
    d = jnp.abs(pred_ref[...] - target_ref[...])
    part = d[:, 0:128]
    for j in range(1, _D // 128):
        part = part + d[:, j * 128:(j + 1) * 128]
    loss_ref[0, 0, :] = jnp.sum(part, axis=1)


def _sc_partial_body(pred_hbm, target_hbm, out_hbm, pbuf, tbuf, lbuf):
    wid = lax.axis_index("s") * 2 + lax.axis_index("c")
    base = _RTC + wid * _RPW

    def chunk(c, carry):
        r0 = (base + c * _CH) * _D
        pltpu.sync_copy(pred_hbm.at[pl.ds(r0, _CH * _D)], pbuf)
        pltpu.sync_copy(target_hbm.at[pl.ds(r0, _CH * _D)], tbuf)
        for r in range(_CH):
            acc = jnp.abs(pbuf[pl.ds(r * _D, _L)] - tbuf[pl.ds(r * _D, _L)])
            for cc in range(1, _D // _L):
                o = r * _D + cc * _L
                acc = acc + jnp.abs(pbuf[pl.ds(o, _L)] - tbuf[pl.ds(o, _L)])
            lbuf[pl.ds((c * _CH + r) * _L, _L)] = acc
        return carry

    lax.fori_loop(0, _RPW // _CH, chunk, 0)
    pltpu.sync_copy(lbuf, out_hbm.at[pl.ds(wid * _RPW * _L, _RPW * _L)])


_sc_partial = functools.partial(
    pl.kernel,
    out_type=jax.ShapeDtypeStruct((_RSC * _L,), jnp.float32),
    mesh=plsc.VectorSubcoreMesh(core_axis_name="c", subcore_axis_name="s"),
    scratch_types=[
        pltpu.VMEM((_CH * _D,), jnp.float32),
        pltpu.VMEM((_CH * _D,), jnp.float32),
        pltpu.VMEM((_RPW * _L,), jnp.float32),
    ],
)(_sc_partial_body)


def _select_body(loss_tc_ref, part_sc_ref, out_ref):
    vh = loss_tc_ref[...]
    vs = jnp.sum(part_sc_ref[...], axis=1)
    bh = lax.bitcast_convert_type(vh, jnp.int32)
    bs = lax.bitcast_convert_type(vs, jnp.int32)

    def count_ge(p):
        return (jnp.sum((bh >= p).astype(jnp.int32)) +
                jnp.sum((bs >= p).astype(jnp.int32)))

    def cond(carry):
        lo, hi = carry
        return lo < hi

    def round_(carry):
        lo, hi = carry
        w = hi - lo + 1
        step = jnp.maximum(w // _WAYS, 1)
        new_lo, new_hi = lo, hi
        for j in range(1, _WAYS):
            p = lo + j * step
            ok = count_ge(p) >= _K
            new_lo = jnp.where(ok, p, new_lo)
            new_hi = jnp.where(ok, new_hi, jnp.minimum(new_hi, p - 1))
        return new_lo, new_hi

    lo0 = jnp.minimum(jnp.min(bh), jnp.min(bs))
    hi0 = jnp.maximum(jnp.max(bh), jnp.max(bs))
    lo, _hi = lax.while_loop(cond, round_, (lo0, hi0))
    t = lax.bitcast_convert_type(lo, jnp.float32)
    gh = bh > lo
    gs = bs > lo
    m = (jnp.sum(gh.astype(jnp.int32)) +
         jnp.sum(gs.astype(jnp.int32))).astype(jnp.float32)
    sum_gt = (jnp.sum(jnp.where(gh, vh, 0.0)) +
              jnp.sum(jnp.where(gs, vs, 0.0)))
    total = sum_gt + (jnp.float32(_K) - m) * t
    out_ref[...] = jnp.full((1, 1), total / jnp.float32(_K), jnp.float32)


def kernel(pred, target):
    p = pred.reshape(_ROWS, _D)
    t = target.reshape(_ROWS, _D)
    loss_tc = pl.pallas_call(
        _tc_loss_body,
        grid=(_NBLK,),
        in_specs=[
            pl.BlockSpec((_BLK, _D), lambda i: (i, 0)),
            pl.BlockSpec((_BLK, _D), lambda i: (i, 0)),
        ],
        out_specs=pl.BlockSpec((1, 1, _BLK), lambda i: (i, 0, 0)),
        out_shape=jax.ShapeDtypeStruct((_NBLK, 1, _BLK), jnp.float32),
    )(p, t)
    part_sc = _sc_partial(pred.reshape(-1), target.reshape(-1))
    out = pl.pallas_call(
        _select_body,
        out_shape=jax.ShapeDtypeStruct((1, 1), jnp.float32),
    )(loss_tc.reshape(_NBLK, _BLK), part_sc.reshape(_RSC, _L))
    return out[0, 0]


# speedup vs baseline: 2.3915x; 2.3915x over previous
import functools

import jax
import jax.numpy as jnp
from jax import lax
from jax.experimental import pallas as pl
from jax.experimental.pallas import tpu as pltpu
from jax.experimental.pallas import tpu_sc as plsc

_ROWS = 4 * 4096
_D = 1024
_K = _ROWS // 2
_BLK = 1024
_WAYS = 8

_NW = 32
_RPW = 64
_RSC = _NW * _RPW
_RTC = _ROWS - _RSC
_NBLK = _RTC // _BLK
_CH = 16
_L = 16


def _tc_loss_body(pred_ref, target_ref, loss_ref):
    d = jnp.abs(pred_ref[...] - target_ref[...])
    part = d[:, 0:128]
    for j in range(1, _D // 128):
        part = part + d[:, j * 128:(j + 1) * 128]
    loss_ref[0, 0, :] = jnp.sum(part, axis=1)


def _sc_partial_body(pred_hbm, target_hbm, out_hbm, pbuf, tbuf, lbuf):
    wid = lax.axis_index("s") * 2 + lax.axis_index("c")
    base = _RTC + wid * _RPW

    def chunk(c, carry):
        r0 = base + c * _CH
        pltpu.sync_copy(pred_hbm.at[pl.ds(r0, _CH), :], pbuf)
        pltpu.sync_copy(target_hbm.at[pl.ds(r0, _CH), :], tbuf)
        for r in range(_CH):
            acc = jnp.abs(pbuf[r, pl.ds(0, _L)] - tbuf[r, pl.ds(0, _L)])
            for cc in range(1, _D // _L):
                o = cc * _L
                acc = acc + jnp.abs(pbuf[r, pl.ds(o, _L)] -
                                    tbuf[r, pl.ds(o, _L)])
            lbuf[pl.ds((c * _CH + r) * _L, _L)] = acc
        return carry

    lax.fori_loop(0, _RPW // _CH, chunk, 0)
    pltpu.sync_copy(lbuf, out_hbm.at[pl.ds(wid * _RPW * _L, _RPW * _L)])


_sc_partial = functools.partial(
    pl.kernel,
    out_type=jax.ShapeDtypeStruct((_RSC * _L,), jnp.float32),
    mesh=plsc.VectorSubcoreMesh(core_axis_name="c", subcore_axis_name="s"),
    scratch_types=[
        pltpu.VMEM((_CH, _D), jnp.float32),
        pltpu.VMEM((_CH, _D), jnp.float32),
        pltpu.VMEM((_RPW * _L,), jnp.float32),
    ],
    compiler_params=pltpu.CompilerParams(use_tc_tiling_on_sc=True),
)(_sc_partial_body)


def _select_body(loss_tc_ref, part_sc_ref, out_ref):
    vh = loss_tc_ref[...]
    vs = jnp.sum(part_sc_ref[...], axis=1)
    bh = lax.bitcast_convert_type(vh, jnp.int32)
    bs = lax.bitcast_convert_type(vs, jnp.int32)

    def count_ge(p):
        return (jnp.sum((bh >= p).astype(jnp.int32)) +
                jnp.sum((bs >= p).astype(jnp.int32)))

    def cond(carry):
        lo, hi = carry
        return lo < hi

    def round_(carry):
        lo, hi = carry
        w = hi - lo + 1
        step = jnp.maximum(w // _WAYS, 1)
        new_lo, new_hi = lo, hi
        for j in range(1, _WAYS):
            p = lo + j * step
            ok = count_ge(p) >= _K
            new_lo = jnp.where(ok, p, new_lo)
            new_hi = jnp.where(ok, new_hi, jnp.minimum(new_hi, p - 1))
        return new_lo, new_hi

    lo0 = jnp.minimum(jnp.min(bh), jnp.min(bs))
    hi0 = jnp.maximum(jnp.max(bh), jnp.max(bs))
    lo, _hi = lax.while_loop(cond, round_, (lo0, hi0))
    t = lax.bitcast_convert_type(lo, jnp.float32)
    gh = bh > lo
    gs = bs > lo
    m = (jnp.sum(gh.astype(jnp.int32)) +
         jnp.sum(gs.astype(jnp.int32))).astype(jnp.float32)
    sum_gt = (jnp.sum(jnp.where(gh, vh, 0.0)) +
              jnp.sum(jnp.where(gs, vs, 0.0)))
    total = sum_gt + (jnp.float32(_K) - m) * t
    out_ref[...] = jnp.full((1, 1), total / jnp.float32(_K), jnp.float32)


def kernel(pred, target):
    p = pred.reshape(_ROWS, _D)
    t = target.reshape(_ROWS, _D)
    loss_tc = pl.pallas_call(
        _tc_loss_body,
        grid=(_NBLK,),
        in_specs=[
            pl.BlockSpec((_BLK, _D), lambda i: (i, 0)),
            pl.BlockSpec((_BLK, _D), lambda i: (i, 0)),
        ],
        out_specs=pl.BlockSpec((1, 1, _BLK), lambda i: (i, 0, 0)),
        out_shape=jax.ShapeDtypeStruct((_NBLK, 1, _BLK), jnp.float32),
    )(p, t)
    part_sc = _sc_partial(p, t)
    out = pl.pallas_call(
        _select_body,
        out_shape=jax.ShapeDtypeStruct((1, 1), jnp.float32),
    )(loss_tc.reshape(_NBLK, _BLK), part_sc.reshape(_RSC, _L))
    return out[0, 0]


# speedup vs baseline: 2.4125x; 1.0088x over previous
import functools

import jax
import jax.numpy as jnp
from jax import lax
from jax.experimental import pallas as pl
from jax.experimental.pallas import tpu as pltpu
from jax.experimental.pallas import tpu_sc as plsc

_ROWS = 4 * 4096
_D = 1024
_K = _ROWS // 2
_BLK = 1024
_WAYS = 8

_NW = 32
_RPW = 64
_RSC = _NW * _RPW
_RTC = _ROWS - _RSC
_NBLK = _RTC // _BLK
_CH = 16
_L = 16


def _tc_loss_body(pred_ref, target_ref, loss_ref):
    d = jnp.abs(pred_ref[...] - target_ref[...])
    part = d[:, 0:128]
    for j in range(1, _D // 128):
        part = part + d[:, j * 128:(j + 1) * 128]
    loss_ref[0, 0, :] = jnp.sum(part, axis=1)


def _sc_partial_body(pred_hbm, target_hbm, out_hbm, pbuf, tbuf, lbuf):
    wid = lax.axis_index("s") * 2 + lax.axis_index("c")
    base = _RTC + wid * _RPW

    def chunk(c, carry):
        r0 = base + c * _CH
        pltpu.sync_copy(pred_hbm.at[pl.ds(r0, _CH), :], pbuf)
        pltpu.sync_copy(target_hbm.at[pl.ds(r0, _CH), :], tbuf)
        for r in range(_CH):
            acc = jnp.abs(pbuf[r, pl.ds(0, _L)] - tbuf[r, pl.ds(0, _L)])
            for cc in range(1, _D // _L):
                o = cc * _L
                acc = acc + jnp.abs(pbuf[r, pl.ds(o, _L)] -
                                    tbuf[r, pl.ds(o, _L)])
            lbuf[pl.ds((c * _CH + r) * _L, _L)] = acc
        return carry

    lax.fori_loop(0, _RPW // _CH, chunk, 0)
    pltpu.sync_copy(lbuf, out_hbm.at[pl.ds(wid * _RPW * _L, _RPW * _L)])


_sc_partial = functools.partial(
    pl.kernel,
    out_type=jax.ShapeDtypeStruct((_RSC * _L,), jnp.float32),
    mesh=plsc.VectorSubcoreMesh(core_axis_name="c", subcore_axis_name="s"),
    scratch_types=[
        pltpu.VMEM((_CH, _D), jnp.float32),
        pltpu.VMEM((_CH, _D), jnp.float32),
        pltpu.VMEM((_RPW * _L,), jnp.float32),
    ],
    compiler_params=pltpu.CompilerParams(use_tc_tiling_on_sc=True),
)(_sc_partial_body)


def _select_body(loss_tc_ref, part_sc_ref, out_ref):
    vh = loss_tc_ref[...]
    vs = jnp.sum(part_sc_ref[...], axis=1)
    bh = lax.bitcast_convert_type(vh, jnp.int32)
    bs = lax.bitcast_convert_type(vs, jnp.int32)

    def count_ge(p):
        return (jnp.sum((bh >= p).astype(jnp.int32)) +
                jnp.sum((bs >= p).astype(jnp.int32)))

    def cond(carry):
        lo, hi = carry
        return lo < hi

    def round_(carry):
        lo, hi = carry
        w = hi - lo + 1
        step = jnp.maximum(w // _WAYS, 1)
        new_lo, new_hi = lo, hi
        for j in range(1, _WAYS):
            p = lo + j * step
            ok = count_ge(p) >= _K
            new_lo = jnp.where(ok, p, new_lo)
            new_hi = jnp.where(ok, new_hi, jnp.minimum(new_hi, p - 1))
        return new_lo, new_hi

    lo0 = jnp.minimum(jnp.min(bh), jnp.min(bs))
    hi0 = jnp.maximum(jnp.max(bh), jnp.max(bs))
    lo, _hi = lax.while_loop(cond, round_, (lo0, hi0))
    t = lax.bitcast_convert_type(lo, jnp.float32)
    gh = bh > lo
    gs = bs > lo
    m = (jnp.sum(gh.astype(jnp.int32)) +
         jnp.sum(gs.astype(jnp.int32))).astype(jnp.float32)
    sum_gt = (jnp.sum(jnp.where(gh, vh, 0.0)) +
              jnp.sum(jnp.where(gs, vs, 0.0)))
    total = sum_gt + (jnp.float32(_K) - m) * t
    out_ref[...] = jnp.full((1, 1), total / jnp.float32(_K), jnp.float32)


def kernel(pred, target):
    p = pred.reshape(_ROWS, _D)
    t = target.reshape(_ROWS, _D)
    part_sc = _sc_partial(p, t)
    loss_tc = pl.pallas_call(
        _tc_loss_body,
        grid=(_NBLK,),
        in_specs=[
            pl.BlockSpec((_BLK, _D), lambda i: (i, 0)),
            pl.BlockSpec((_BLK, _D), lambda i: (i, 0)),
        ],
        out_specs=pl.BlockSpec((1, 1, _BLK), lambda i: (i, 0, 0)),
        out_shape=jax.ShapeDtypeStruct((_NBLK, 1, _BLK), jnp.float32),
    )(p, t)
    out = pl.pallas_call(
        _select_body,
        out_shape=jax.ShapeDtypeStruct((1, 1), jnp.float32),
    )(loss_tc.reshape(_NBLK, _BLK), part_sc.reshape(_RSC, _L))
    return out[0, 0]


# speedup vs baseline: 4.2135x; 1.7465x over previous
import jax
import jax.numpy as jnp
from jax import lax
from jax.experimental import pallas as pl
from jax.experimental.pallas import tpu as pltpu

_ROWS = 4 * 4096
_D = 1024
_K = _ROWS // 2
_BLK = 1024
_NBLK = _ROWS // _BLK
_WAYS = 8


def _topk_l1_body(pred_ref, target_ref, out_ref, loss_ref):
    i = pl.program_id(0)
    d = jnp.abs(pred_ref[...] - target_ref[...])
    part = d[:, 0:128]
    for j in range(1, _D // 128):
        part = part + d[:, j * 128:(j + 1) * 128]
    loss_ref[i, :] = jnp.sum(part, axis=1)

    @pl.when(i == _NBLK - 1)
    def _():
        v = loss_ref[...]
        bits = lax.bitcast_convert_type(v, jnp.int32)

        def cond(carry):
            lo, hi = carry
            return lo < hi

        def round_(carry):
            lo, hi = carry
            w = hi - lo + 1
            step = jnp.maximum(w // _WAYS, 1)
            new_lo, new_hi = lo, hi
            for j in range(1, _WAYS):
                p = lo + j * step
                cnt = jnp.sum((bits >= p).astype(jnp.int32))
                ok = cnt >= _K
                new_lo = jnp.where(ok, p, new_lo)
                new_hi = jnp.where(ok, new_hi, jnp.minimum(new_hi, p - 1))
            return new_lo, new_hi

        lo0 = jnp.min(bits)
        hi0 = jnp.max(bits)
        lo, _hi = lax.while_loop(cond, round_, (lo0, hi0))
        t = lax.bitcast_convert_type(lo, jnp.float32)
        gt = bits > lo
        m = jnp.sum(gt.astype(jnp.int32)).astype(jnp.float32)
        sum_gt = jnp.sum(jnp.where(gt, v, 0.0))
        total = sum_gt + (jnp.float32(_K) - m) * t
        out_ref[...] = jnp.full((1, 1), total / jnp.float32(_K), jnp.float32)


def kernel(pred, target):
    p = pred.reshape(_ROWS, _D)
    t = target.reshape(_ROWS, _D)
    out = pl.pallas_call(
        _topk_l1_body,
        grid=(_NBLK,),
        in_specs=[
            pl.BlockSpec((_BLK, _D), lambda i: (i, 0)),
            pl.BlockSpec((_BLK, _D), lambda i: (i, 0)),
        ],
        out_specs=pl.BlockSpec((1, 1), lambda i: (0, 0)),
        out_shape=jax.ShapeDtypeStruct((1, 1), jnp.float32),
        scratch_shapes=[pltpu.VMEM((_NBLK, _BLK), jnp.float32)],
    )(p, t)
    return out[0, 0]
